# baseline (speedup 1.0000x reference)
import functools

import jax
import jax.numpy as jnp
from jax import lax
from jax.experimental import pallas as pl
from jax.experimental.pallas import tpu as pltpu
from jax.experimental.pallas import tpu_sc as plsc


_TC_STEPS = 10

_SC_WORKERS = 32
_SC_CHUNK_ROWS = 400


def _copy2_kernel(u_ref, i_ref, ou_ref, oi_ref):
    ou_ref[...] = u_ref[...]
    oi_ref[...] = i_ref[...]


def _tc_copy2(embed_user, embed_item):
    nu, d = embed_user.shape
    ni, _ = embed_item.shape
    bu, bi = nu // _TC_STEPS, ni // _TC_STEPS

    def spec(block_rows):
        return pl.BlockSpec((block_rows, d), lambda s: (s, 0))

    return pl.pallas_call(
        _copy2_kernel,
        grid=(_TC_STEPS,),
        compiler_params=pltpu.CompilerParams(dimension_semantics=("parallel",)),
        in_specs=[spec(bu), spec(bi)],
        out_specs=[spec(bu), spec(bi)],
        out_shape=[
            jax.ShapeDtypeStruct(embed_user.shape, embed_user.dtype),
            jax.ShapeDtypeStruct(embed_item.shape, embed_item.dtype),
        ],
    )(embed_user, embed_item)


def _sc_copy(embed_tag):
    n, d = embed_tag.shape
    n_chunks = n // _SC_CHUNK_ROWS
    chunks_per_worker = -(-n_chunks // _SC_WORKERS)
    mesh = plsc.VectorSubcoreMesh(core_axis_name="c", subcore_axis_name="s")

    @functools.partial(
        pl.kernel,
        mesh=mesh,
        out_type=jax.ShapeDtypeStruct((n, d), embed_tag.dtype),
        scratch_types=[pltpu.VMEM((_SC_CHUNK_ROWS, d), embed_tag.dtype)],
    )
    def sc_tag_copy(tag_hbm, out_hbm, buf):
        wid = lax.axis_index("s") * 2 + lax.axis_index("c")
        for c in range(chunks_per_worker):
            chunk = wid + _SC_WORKERS * c

            @pl.when(chunk < n_chunks)
            def _():
                base = chunk * _SC_CHUNK_ROWS
                pltpu.sync_copy(tag_hbm.at[pl.ds(base, _SC_CHUNK_ROWS)], buf)
                pltpu.sync_copy(buf, out_hbm.at[pl.ds(base, _SC_CHUNK_ROWS)])

    return sc_tag_copy(embed_tag)


def kernel(embed_user, embed_item, embed_tag):
    out_tag = _sc_copy(embed_tag)
    out_user, out_item = _tc_copy2(embed_user, embed_item)
    return (out_user, out_item, out_tag)
